# Initial kernel scaffold; baseline (speedup 1.0000x reference)
import jax
import jax.numpy as jnp
from jax.experimental import pallas as pl


def kernel(indices, offsets, W):
    raise NotImplementedError("write your pallas kernel here")


# speedup vs baseline: 1.0164x; 1.0164x over previous
import functools

import jax
import jax.numpy as jnp
from jax import lax
from jax.experimental import pallas as pl
from jax.experimental.pallas import tpu as pltpu
from jax.experimental.pallas import tpu_sc as plsc

N_IDX = 819200
BATCH = 16384
D = 64
LANES = 128
ROWS = N_IDX // LANES
PAD_ROWS = 64
NC = 2
NS = 16
CH = 4
CHROWS = CH * LANES
HALF = BATCH // NC
TRASH = HALF
ACC_ROWS = HALF + 8


def _sc_body(idx_hbm, seg_hbm, off_hbm, w_hbm, out_hbm,
             idx_v, seg_v, rows_v, off_v, acc, sem0, sem1):
    c = lax.axis_index("c")
    s = lax.axis_index("s")
    sems = (sem0, sem1)

    zero = jnp.zeros((16,), jnp.float32)

    def zbody(i, carry):
        for q in range(D // 16):
            rows_v[0, i, pl.ds(q * 16, 16)] = zero
        return carry

    lax.fori_loop(0, CHROWS, zbody, 0)
    pltpu.sync_copy(rows_v.at[0], acc.at[pl.ds(s * CHROWS, CHROWS)])
    plsc.subcore_barrier()

    pltpu.sync_copy(off_hbm.at[pl.ds(c * HALF, 8)], off_v.at[pl.ds(0, 8)])
    pltpu.sync_copy(off_hbm.at[pl.ds((c + 1) * HALF, 8)], off_v.at[pl.ds(8, 8)])
    offv = off_v[pl.ds(0, 16)]
    p0 = offv[0]
    p1 = offv[8]
    r0 = p0 // LANES
    r1 = (p1 + LANES - 1) // LANES
    nrows = r1 - r0
    nr = ((nrows + NS * CH - 1) // (NS * CH)) * CH
    rlo = r0 + s * nr
    nch = nr // CH

    seg_base = c * HALF

    def stage_fire(g, b):
        row = rlo + g * CH
        pltpu.sync_copy(idx_hbm.at[pl.ds(row, CH)], idx_v.at[b])
        pltpu.sync_copy(seg_hbm.at[pl.ds(row, CH)], seg_v.at[b])
        for j in range(CH):
            pltpu.async_copy(w_hbm.at[idx_v.at[b, j]],
                             rows_v.at[b, pl.ds(j * LANES, LANES)],
                             sems[b])

    @pl.when(nch > 0)
    def _():
        stage_fire(0, 0)

    def tbody(t, carry):
        for b in range(2):
            g = 2 * t + b

            @pl.when(g < nch)
            def _():
                @pl.when(g + 1 < nch)
                def _():
                    stage_fire(g + 1, 1 - b)

                for j in range(CH):
                    for q in range(LANES // 16):
                        v = seg_v[b, j, pl.ds(q * 16, 16)] - seg_base
                        bad = (v < 0) | (v >= HALF)
                        seg_v[b, j, pl.ds(q * 16, 16)] = jnp.where(
                            bad, TRASH, v)

                pltpu.make_async_copy(w_hbm.at[pl.ds(0, CHROWS)],
                                      rows_v.at[b], sems[b]).wait()

                for j in range(CH):
                    pltpu.sync_copy(rows_v.at[b, pl.ds(j * LANES, LANES)],
                                    acc.at[seg_v.at[b, j]], add=True)
        return carry

    lax.fori_loop(0, (nch + 1) // 2, tbody, 0)

    plsc.subcore_barrier()
    pltpu.sync_copy(acc.at[pl.ds(s * CHROWS, CHROWS)], rows_v.at[0])
    pltpu.sync_copy(rows_v.at[0],
                    out_hbm.at[pl.ds(c * HALF + s * CHROWS, CHROWS)])


_sc_bag = functools.partial(
    pl.kernel,
    mesh=plsc.VectorSubcoreMesh(core_axis_name="c", subcore_axis_name="s"),
    out_type=jax.ShapeDtypeStruct((BATCH, D), jnp.float32),
    compiler_params=pltpu.CompilerParams(use_tc_tiling_on_sc=False),
    scratch_types=[
        pltpu.VMEM((2, CH, LANES), jnp.int32),
        pltpu.VMEM((2, CH, LANES), jnp.int32),
        pltpu.VMEM((2, CHROWS, D), jnp.float32),
        pltpu.VMEM((16,), jnp.int32),
        pltpu.VMEM_SHARED((ACC_ROWS, D), jnp.float32),
        pltpu.SemaphoreType.DMA,
        pltpu.SemaphoreType.DMA,
    ],
)(_sc_body)


def kernel(indices, offsets, W):
    seg = (jnp.searchsorted(offsets,
                            jnp.arange(N_IDX, dtype=offsets.dtype),
                            side="right") - 1).astype(jnp.int32)
    idx2 = jnp.zeros((ROWS + PAD_ROWS, LANES), jnp.int32)
    idx2 = idx2.at[:ROWS].set(indices.reshape(ROWS, LANES))
    seg2 = jnp.full((ROWS + PAD_ROWS, LANES), BATCH, jnp.int32)
    seg2 = seg2.at[:ROWS].set(seg.reshape(ROWS, LANES))
    off_pad = jnp.concatenate(
        [offsets.astype(jnp.int32),
         jnp.full((16,), N_IDX, jnp.int32)])
    return _sc_bag(idx2, seg2, off_pad, W)


# speedup vs baseline: 123.0830x; 121.0923x over previous
import functools

import jax
import jax.numpy as jnp
from jax import lax
from jax.experimental import pallas as pl
from jax.experimental.pallas import tpu as pltpu
from jax.experimental.pallas import tpu_sc as plsc

N_IDX = 819200
BATCH = 16384
D = 64
LANES = 128
ROWS = N_IDX // LANES
NC = 2
NS = 16
CH = 4
CHROWS = CH * LANES
HALF = BATCH // NC
TRASH = HALF
ACC_ROWS = HALF + 8
OFF_V = HALF + 32
BS_HI = HALF + 16
INT_MAX = 2**31 - 1


def _sc_body(idx_hbm, off_hbm, w_hbm, out_hbm,
             idx_v, seg_v, rows_v, off_half, acc, sem0, sem1):
    c = lax.axis_index("c")
    s = lax.axis_index("s")
    sems = (sem0, sem1)

    zero = jnp.zeros((16,), jnp.float32)

    def zbody(i, carry):
        for q in range(D // 16):
            rows_v[0, i, pl.ds(q * 16, 16)] = zero
        return carry

    lax.fori_loop(0, CHROWS, zbody, 0)
    pltpu.sync_copy(rows_v.at[0], acc.at[pl.ds(s * CHROWS, CHROWS)])
    plsc.subcore_barrier()

    for q in range((OFF_V - HALF) // 16):
        off_half[pl.ds(HALF + q * 16, 16)] = jnp.full((16,), INT_MAX,
                                                      jnp.int32)
    pltpu.sync_copy(off_hbm.at[pl.ds(c * HALF, HALF)],
                    off_half.at[pl.ds(0, HALF)])

    @pl.when(c == 0)
    def _():
        pltpu.sync_copy(off_hbm.at[pl.ds(HALF, 8)],
                        off_half.at[pl.ds(HALF, 8)])

    p1 = jnp.minimum(off_half[pl.ds(HALF, 16)][0], N_IDX)
    p0 = off_half[pl.ds(0, 16)][0]
    r0 = p0 // LANES
    r1 = (p1 + LANES - 1) // LANES
    nrows = r1 - r0

    q_, rem = nrows // NS, nrows % NS
    lo = r0 + s * q_ + jnp.minimum(s, rem)
    nr_w = q_ + jnp.where(s < rem, 1, 0)
    hi = lo + nr_w
    nch = (nr_w + CH - 1) // CH

    lanes16 = lax.iota(jnp.int32, 16)

    def stage_fire(g, b):
        row0 = lo + g * CH
        for j in range(CH):
            @pl.when(row0 + j < hi)
            def _(j=j):
                pltpu.sync_copy(idx_hbm.at[pl.ds((row0 + j) * LANES, LANES)],
                                idx_v.at[b, pl.ds(j * LANES, LANES)])
                pltpu.async_copy(
                    w_hbm.at[idx_v.at[b, pl.ds(j * LANES, LANES)]],
                    rows_v.at[b, pl.ds(j * LANES, LANES)],
                    sems[b])

    def process(g, b):
        row0 = lo + g * CH
        for j in range(CH):
            def qbody(qq, carry, j=j):
                target = (row0 + j) * LANES + qq * 16 + 1 + lanes16
                blo = jnp.full((16,), -1, jnp.int32)
                bhi = jnp.full((16,), BS_HI, jnp.int32)
                for _ in range(14):
                    mid = jnp.maximum((blo + bhi) // 2, 0)
                    v = plsc.load_gather(off_half, [mid])
                    pred = v < target
                    blo = jnp.where(pred, mid, blo)
                    bhi = jnp.where(pred, bhi, mid)
                seg = bhi - 1
                seg = jnp.where((seg < 0) | (seg >= HALF), TRASH, seg)
                seg_v[b, j, pl.ds(qq * 16, 16)] = seg
                return carry

            lax.fori_loop(0, LANES // 16, qbody, 0)

        @pl.when(g + 1 < nch)
        def _():
            stage_fire(g + 1, 1 - b)

        for j in range(CH):
            @pl.when(row0 + j < hi)
            def _(j=j):
                pltpu.make_async_copy(
                    w_hbm.at[pl.ds(0, LANES)],
                    rows_v.at[b, pl.ds(j * LANES, LANES)],
                    sems[b]).wait()
                pltpu.sync_copy(rows_v.at[b, pl.ds(j * LANES, LANES)],
                                acc.at[seg_v.at[b, j]], add=True)

    @pl.when(nch > 0)
    def _():
        stage_fire(0, 0)

    def tbody(t, carry):
        for b in range(2):
            g = 2 * t + b

            @pl.when(g < nch)
            def _(g=g, b=b):
                process(g, b)
        return carry

    lax.fori_loop(0, (nch + 1) // 2, tbody, 0)

    plsc.subcore_barrier()
    pltpu.sync_copy(acc.at[pl.ds(s * CHROWS, CHROWS)], rows_v.at[0])
    pltpu.sync_copy(rows_v.at[0],
                    out_hbm.at[pl.ds(c * HALF + s * CHROWS, CHROWS)])


_sc_bag = functools.partial(
    pl.kernel,
    mesh=plsc.VectorSubcoreMesh(core_axis_name="c", subcore_axis_name="s"),
    out_type=jax.ShapeDtypeStruct((BATCH, D), jnp.float32),
    compiler_params=pltpu.CompilerParams(use_tc_tiling_on_sc=False,
                                         needs_layout_passes=False),
    scratch_types=[
        pltpu.VMEM((2, CHROWS), jnp.int32),
        pltpu.VMEM((2, CH, LANES), jnp.int32),
        pltpu.VMEM((2, CHROWS, D), jnp.float32),
        pltpu.VMEM((OFF_V,), jnp.int32),
        pltpu.VMEM_SHARED((ACC_ROWS, D), jnp.float32),
        pltpu.SemaphoreType.DMA,
        pltpu.SemaphoreType.DMA,
    ],
)(_sc_body)


def kernel(indices, offsets, W):
    return _sc_bag(indices, offsets, W)


# speedup vs baseline: 144.6547x; 1.1753x over previous
import functools

import jax
import jax.numpy as jnp
from jax import lax
from jax.experimental import pallas as pl
from jax.experimental.pallas import tpu as pltpu
from jax.experimental.pallas import tpu_sc as plsc

N_IDX = 819200
BATCH = 16384
D = 64
LANES = 128
ROWS = N_IDX // LANES
NC = 2
NS = 16
CH = 4
CHROWS = CH * LANES
HALF = BATCH // NC
TRASH = HALF
ACC_ROWS = HALF + 8
OFF_V = HALF + 32
BS_HI = HALF + 16
INT_MAX = 2**31 - 1


def _sc_body(idx_hbm, off_hbm, w_hbm, out_hbm,
             idx_v, seg_v, rows_v, off_half, acc, sem0, sem1):
    c = lax.axis_index("c")
    s = lax.axis_index("s")
    sems = (sem0, sem1)

    zero = jnp.zeros((16,), jnp.float32)

    def zbody(i, carry):
        for q in range(D // 16):
            rows_v[0, i, pl.ds(q * 16, 16)] = zero
        return carry

    lax.fori_loop(0, CHROWS, zbody, 0)
    pltpu.sync_copy(rows_v.at[0], acc.at[pl.ds(s * CHROWS, CHROWS)])
    plsc.subcore_barrier()

    for q in range((OFF_V - HALF) // 16):
        off_half[pl.ds(HALF + q * 16, 16)] = jnp.full((16,), INT_MAX,
                                                      jnp.int32)
    pltpu.sync_copy(off_hbm.at[pl.ds(c * HALF, HALF)],
                    off_half.at[pl.ds(0, HALF)])

    @pl.when(c == 0)
    def _():
        pltpu.sync_copy(off_hbm.at[pl.ds(HALF, 8)],
                        off_half.at[pl.ds(HALF, 8)])

    p1 = jnp.minimum(off_half[pl.ds(HALF, 16)][0], N_IDX)
    p0 = off_half[pl.ds(0, 16)][0]
    r0 = p0 // LANES
    r1 = (p1 + LANES - 1) // LANES
    nrows = r1 - r0

    q_, rem = nrows // NS, nrows % NS
    lo = r0 + s * q_ + jnp.minimum(s, rem)
    nr_w = q_ + jnp.where(s < rem, 1, 0)
    hi = lo + nr_w
    nch = (nr_w + CH - 1) // CH

    lanes16 = lax.iota(jnp.int32, 16)

    def stage_fire(g, b):
        row0 = lo + g * CH

        @pl.when(row0 + CH <= hi)
        def _():
            pltpu.sync_copy(idx_hbm.at[pl.ds(row0 * LANES, CHROWS)],
                            idx_v.at[b])
            for j in range(CH):
                pltpu.async_copy(
                    w_hbm.at[idx_v.at[b, pl.ds(j * LANES, LANES)]],
                    rows_v.at[b, pl.ds(j * LANES, LANES)],
                    sems[b])

        @pl.when(row0 + CH > hi)
        def _():
            for j in range(CH):
                @pl.when(row0 + j < hi)
                def _(j=j):
                    pltpu.sync_copy(
                        idx_hbm.at[pl.ds((row0 + j) * LANES, LANES)],
                        idx_v.at[b, pl.ds(j * LANES, LANES)])
                    pltpu.async_copy(
                        w_hbm.at[idx_v.at[b, pl.ds(j * LANES, LANES)]],
                        rows_v.at[b, pl.ds(j * LANES, LANES)],
                        sems[b])

    def process(g, b):
        row0 = lo + g * CH
        for j in range(CH):
            def qbody(qq, carry, j=j):
                base = (row0 + j) * LANES + qq * 32 + 1
                tg = [base + lanes16, base + 16 + lanes16]
                blo = [jnp.full((16,), -1, jnp.int32)] * 2
                bhi = [jnp.full((16,), BS_HI, jnp.int32)] * 2
                for _ in range(14):
                    for u in range(2):
                        mid = jnp.maximum((blo[u] + bhi[u]) // 2, 0)
                        v = plsc.load_gather(off_half, [mid])
                        pred = v < tg[u]
                        blo[u] = jnp.where(pred, mid, blo[u])
                        bhi[u] = jnp.where(pred, bhi[u], mid)
                for u in range(2):
                    seg = bhi[u] - 1
                    seg = jnp.where((seg < 0) | (seg >= HALF), TRASH, seg)
                    seg_v[b, j, pl.ds(qq * 32 + u * 16, 16)] = seg
                return carry

            lax.fori_loop(0, LANES // 32, qbody, 0)

        @pl.when(g + 1 < nch)
        def _():
            stage_fire(g + 1, 1 - b)

        @pl.when(row0 + CH <= hi)
        def _():
            pltpu.make_async_copy(w_hbm.at[pl.ds(0, CHROWS)],
                                  rows_v.at[b], sems[b]).wait()
            for j in range(CH):
                pltpu.sync_copy(rows_v.at[b, pl.ds(j * LANES, LANES)],
                                acc.at[seg_v.at[b, j]], add=True)

        @pl.when(row0 + CH > hi)
        def _():
            for j in range(CH):
                @pl.when(row0 + j < hi)
                def _(j=j):
                    pltpu.make_async_copy(
                        w_hbm.at[pl.ds(0, LANES)],
                        rows_v.at[b, pl.ds(j * LANES, LANES)],
                        sems[b]).wait()
                    pltpu.sync_copy(rows_v.at[b, pl.ds(j * LANES, LANES)],
                                    acc.at[seg_v.at[b, j]], add=True)

    @pl.when(nch > 0)
    def _():
        stage_fire(0, 0)

    def tbody(t, carry):
        for b in range(2):
            g = 2 * t + b

            @pl.when(g < nch)
            def _(g=g, b=b):
                process(g, b)
        return carry

    lax.fori_loop(0, (nch + 1) // 2, tbody, 0)

    plsc.subcore_barrier()
    pltpu.sync_copy(acc.at[pl.ds(s * CHROWS, CHROWS)], rows_v.at[0])
    pltpu.sync_copy(rows_v.at[0],
                    out_hbm.at[pl.ds(c * HALF + s * CHROWS, CHROWS)])


_sc_bag = functools.partial(
    pl.kernel,
    mesh=plsc.VectorSubcoreMesh(core_axis_name="c", subcore_axis_name="s"),
    out_type=jax.ShapeDtypeStruct((BATCH, D), jnp.float32),
    compiler_params=pltpu.CompilerParams(use_tc_tiling_on_sc=False,
                                         needs_layout_passes=False),
    scratch_types=[
        pltpu.VMEM((2, CHROWS), jnp.int32),
        pltpu.VMEM((2, CH, LANES), jnp.int32),
        pltpu.VMEM((2, CHROWS, D), jnp.float32),
        pltpu.VMEM((OFF_V,), jnp.int32),
        pltpu.VMEM_SHARED((ACC_ROWS, D), jnp.float32),
        pltpu.SemaphoreType.DMA,
        pltpu.SemaphoreType.DMA,
    ],
)(_sc_body)


def kernel(indices, offsets, W):
    return _sc_bag(indices, offsets, W)
